# baseline (speedup 1.0000x reference)
import functools
import math

import jax
import jax.numpy as jnp
from jax import lax
from jax.experimental import pallas as pl
from jax.experimental.pallas import tpu as pltpu
from jax.experimental.pallas import tpu_sc as plsc

_N = 10000
_E = 160000
_D = 128
_A = 16
_CE = 4
_CF = 8
_H = 64
_NUM_AVG_NEIGHBORS = 16.0

_BN = 2000


def _h_body(nf_ref, wup_ref, h_ref):
    h_ref[...] = jnp.dot(nf_ref[...], wup_ref[...],
                         preferred_element_type=jnp.float32) * (1.0 / math.sqrt(_D))


def _h_call(node_feats, W_up):
    return pl.pallas_call(
        _h_body,
        grid=(_N // _BN,),
        in_specs=[
            pl.BlockSpec((_BN, _D), lambda i: (i, 0)),
            pl.BlockSpec((_D, _D), lambda i: (0, 0)),
        ],
        out_specs=pl.BlockSpec((_BN, _D), lambda i: (i, 0)),
        out_shape=jax.ShapeDtypeStruct((_N, _D), jnp.float32),
    )(node_feats, W_up)


_BE = 2000


def _edge_body(ef_ref, er_ref, w1_ref, w2_ref, w3_ref, w4c_ref, acc_ref):
    t = jax.nn.silu(jnp.dot(ef_ref[...], w1_ref[...],
                            preferred_element_type=jnp.float32) * (1.0 / math.sqrt(_CF)))
    t = jax.nn.silu(jnp.dot(t, w2_ref[...],
                            preferred_element_type=jnp.float32) * (1.0 / math.sqrt(_H)))
    t = jax.nn.silu(jnp.dot(t, w3_ref[...],
                            preferred_element_type=jnp.float32) * (1.0 / math.sqrt(_H)))
    er = er_ref[...]
    t4 = jnp.concatenate([t * er[:, v:v + 1] for v in range(_CE)], axis=1)
    acc_ref[...] = jnp.dot(t4, w4c_ref[...],
                           preferred_element_type=jnp.float32) * (
        1.0 / (math.sqrt(_H) * math.sqrt(_CE)))


def _edge_call(edge_feats, edge_attrs_real, W1, W2, W3, W4c):
    return pl.pallas_call(
        _edge_body,
        grid=(_E // _BE,),
        in_specs=[
            pl.BlockSpec((_BE, _CF), lambda i: (i, 0)),
            pl.BlockSpec((_BE, _CE), lambda i: (i, 0)),
            pl.BlockSpec((_CF, _H), lambda i: (0, 0)),
            pl.BlockSpec((_H, _H), lambda i: (0, 0)),
            pl.BlockSpec((_H, _H), lambda i: (0, 0)),
            pl.BlockSpec((_CE * _H, _D), lambda i: (0, 0)),
        ],
        out_specs=pl.BlockSpec((_BE, _D), lambda i: (i, 0)),
        out_shape=jax.ShapeDtypeStruct((_E, _D), jnp.float32),
        compiler_params=pltpu.CompilerParams(dimension_semantics=("parallel",)),
    )(edge_feats, edge_attrs_real, W1, W2, W3, W4c)


_NC = 2
_NS = 16
_NW = _NC * _NS
_CH = 128
_NCHUNKS = _E // _CH
_CPW = -(-_NCHUNKS // _NW)
_RPT = 632
_RPT_LAST = _N - 15 * _RPT
_RZB = 8


def _sc_body(h_hbm, acc_hbm, snd_hbm, rcv_hbm, out_hbm,
             sidx_v, ridx_v, hrows_v, arows_v, zbuf_v, msh, sem0, sem1):
    cid = lax.axis_index("c")
    sid = lax.axis_index("s")
    wid = sid * _NC + cid
    sems = (sem0, sem1)

    def _zfill(i, carry):
        r = i // (_D // 16)
        c = (i % (_D // 16)) * 16
        zbuf_v[r, pl.ds(c, 16)] = jnp.zeros((16,), jnp.float32)
        return carry

    lax.fori_loop(0, _RZB * (_D // 16), _zfill, 0)

    row0 = sid * _RPT
    nrows = jnp.where(sid == _NS - 1, _RPT_LAST, _RPT)

    def _zcopy(m, carry):
        pltpu.sync_copy(zbuf_v, msh.at[pl.ds(row0 + m * _RZB, _RZB)])
        return carry

    lax.fori_loop(0, nrows // _RZB, _zcopy, 0)

    def _issue(k, b):
        c = k * _NW + wid

        @pl.when(c < _NCHUNKS)
        def _():
            base = c * _CH
            pltpu.sync_copy(snd_hbm.at[pl.ds(base, _CH)], sidx_v.at[b])
            pltpu.sync_copy(rcv_hbm.at[pl.ds(base, _CH)], ridx_v.at[b])
            pltpu.async_copy(h_hbm.at[sidx_v.at[b]], hrows_v.at[b], sems[b])

    plsc.subcore_barrier()
    _issue(0, 0)

    def _pair(p, carry):
        for b in range(2):
            k = p * 2 + b
            c = k * _NW + wid

            @pl.when(c < _NCHUNKS)
            def _():
                pltpu.make_async_copy(h_hbm.at[pl.ds(0, _CH)], hrows_v.at[b],
                                      sems[b]).wait()
                _issue(k + 1, 1 - b)
                pltpu.sync_copy(acc_hbm.at[pl.ds(c * _CH, _CH)], arows_v)

                hb = hrows_v.at[b]
                ab = arows_v

                def _mul(q, carry2):
                    for rr in range(4):
                        r = q * 4 + rr
                        for j in range(_D // 16):
                            col = j * 16
                            hb[r, pl.ds(col, 16)] = (
                                hb[r, pl.ds(col, 16)] * ab[r, pl.ds(col, 16)])
                    return carry2

                lax.fori_loop(0, _CH // 4, _mul, 0)
                pltpu.sync_copy(hb, msh.at[ridx_v.at[b]], add=True)

        return carry

    lax.fori_loop(0, _CPW // 2, _pair, 0)
    plsc.subcore_barrier()

    @pl.when(sid < _NS - 1)
    def _():
        pltpu.sync_copy(msh.at[pl.ds(row0, _RPT)],
                        out_hbm.at[cid, pl.ds(row0, _RPT)])

    @pl.when(sid == _NS - 1)
    def _():
        pltpu.sync_copy(msh.at[pl.ds(row0, _RPT_LAST)],
                        out_hbm.at[cid, pl.ds(row0, _RPT_LAST)])


def _sc_call(h, acc, sender, receiver):
    mesh = plsc.VectorSubcoreMesh(core_axis_name="c", subcore_axis_name="s")
    f = pl.kernel(
        _sc_body,
        mesh=mesh,
        out_type=jax.ShapeDtypeStruct((_NC, _N, _D), jnp.float32),
        scratch_types=[
            pltpu.VMEM((2, _CH), jnp.int32),
            pltpu.VMEM((2, _CH), jnp.int32),
            pltpu.VMEM((2, _CH, _D), jnp.float32),
            pltpu.VMEM((_CH, _D), jnp.float32),
            pltpu.VMEM((_RZB, _D), jnp.float32),
            pltpu.VMEM_SHARED((_N, _D), jnp.float32),
            pltpu.SemaphoreType.DMA,
            pltpu.SemaphoreType.DMA,
        ],
    )
    return f(h, acc, sender, receiver)


def _post_body(mp_ref, nf_ref, na_ref, wlin_ref, wskip_ref, mr_ref, mi_ref):
    m = mp_ref[0] + mp_ref[1]
    nf = nf_ref[...]
    na = na_ref[...]
    sc = jnp.zeros((_BN, _D), jnp.float32)
    for v in range(_A):
        sc = sc + na[:, v:v + 1] * jnp.dot(nf, wskip_ref[:, v, :],
                                           preferred_element_type=jnp.float32)
    sc = sc * (1.0 / math.sqrt(_D * _A))
    s = 1.0 / (math.sqrt(_D) * 2.0 * _NUM_AVG_NEIGHBORS)
    mr = jnp.dot(m, wlin_ref[...], preferred_element_type=jnp.float32) * s + sc
    mr = jax.nn.silu(mr)
    mi = jax.nn.silu(jnp.dot(mr, wlin_ref[...],
                             preferred_element_type=jnp.float32) * s)
    mr_ref[...] = mr
    mi_ref[...] = mi


def _post_call(mp, node_feats, node_attrs, W_lin, W_skip):
    return pl.pallas_call(
        _post_body,
        grid=(_N // _BN,),
        in_specs=[
            pl.BlockSpec((_NC, _BN, _D), lambda i: (0, i, 0)),
            pl.BlockSpec((_BN, _D), lambda i: (i, 0)),
            pl.BlockSpec((_BN, _A), lambda i: (i, 0)),
            pl.BlockSpec((_D, _D), lambda i: (0, 0)),
            pl.BlockSpec((_D, _A, _D), lambda i: (0, 0, 0)),
        ],
        out_specs=[
            pl.BlockSpec((_BN, _D), lambda i: (i, 0)),
            pl.BlockSpec((_BN, _D), lambda i: (i, 0)),
        ],
        out_shape=[
            jax.ShapeDtypeStruct((_N, _D), jnp.float32),
            jax.ShapeDtypeStruct((_N, _D), jnp.float32),
        ],
        compiler_params=pltpu.CompilerParams(dimension_semantics=("parallel",)),
    )(mp, node_feats, node_attrs, W_lin, W_skip)


def kernel(node_attrs, node_feats, edge_attrs_real, edge_attrs_imag, edge_feats,
           edge_index, W_up, W1, W2, W3, W4, W_lin, W_skip):
    del edge_attrs_imag
    W4c = W4.reshape(_H, _D, _CE).transpose(2, 0, 1).reshape(_CE * _H, _D)
    sender = edge_index[0]
    receiver = edge_index[1]
    h = _h_call(node_feats, W_up)
    acc = _edge_call(edge_feats, edge_attrs_real, W1, W2, W3, W4c)
    mp = _sc_call(h, acc, sender, receiver)
    mr, mi = _post_call(mp, node_feats, node_attrs, W_lin, W_skip)
    return jnp.stack((mr, mi), axis=-1).reshape(_N, _D, 1, 2)


# speedup vs baseline: 1.0030x; 1.0030x over previous
import functools
import math

import jax
import jax.numpy as jnp
from jax import lax
from jax.experimental import pallas as pl
from jax.experimental.pallas import tpu as pltpu
from jax.experimental.pallas import tpu_sc as plsc

_N = 10000
_E = 160000
_D = 128
_A = 16
_CE = 4
_CF = 8
_H = 64
_NUM_AVG_NEIGHBORS = 16.0

_BN = 2000


def _h_body(nf_ref, wup_ref, h_ref):
    h_ref[...] = jnp.dot(nf_ref[...], wup_ref[...],
                         preferred_element_type=jnp.float32) * (1.0 / math.sqrt(_D))


def _h_call(node_feats, W_up):
    return pl.pallas_call(
        _h_body,
        grid=(_N // _BN,),
        in_specs=[
            pl.BlockSpec((_BN, _D), lambda i: (i, 0)),
            pl.BlockSpec((_D, _D), lambda i: (0, 0)),
        ],
        out_specs=pl.BlockSpec((_BN, _D), lambda i: (i, 0)),
        out_shape=jax.ShapeDtypeStruct((_N, _D), jnp.float32),
    )(node_feats, W_up)


_BE = 2000


def _edge_body(ef_ref, er_ref, w1_ref, w2_ref, w3_ref, w4w_ref, s_ref, acc_ref):
    t = jax.nn.silu(jnp.dot(ef_ref[...], w1_ref[...],
                            preferred_element_type=jnp.float32) * (1.0 / math.sqrt(_CF)))
    t = jax.nn.silu(jnp.dot(t, w2_ref[...],
                            preferred_element_type=jnp.float32) * (1.0 / math.sqrt(_H)))
    t = jax.nn.silu(jnp.dot(t, w3_ref[...],
                            preferred_element_type=jnp.float32) * (1.0 / math.sqrt(_H)))
    u = jnp.dot(t, w4w_ref[...], preferred_element_type=jnp.float32)
    erb = jnp.dot(er_ref[...], s_ref[...], preferred_element_type=jnp.float32)
    p = u * erb
    acc = p[:, 0:_D]
    for v in range(1, _CE):
        acc = acc + p[:, v * _D:(v + 1) * _D]
    acc_ref[...] = acc * (1.0 / (math.sqrt(_H) * math.sqrt(_CE)))


def _edge_call(edge_feats, edge_attrs_real, W1, W2, W3, W4w, S):
    return pl.pallas_call(
        _edge_body,
        grid=(_E // _BE,),
        in_specs=[
            pl.BlockSpec((_BE, _CF), lambda i: (i, 0)),
            pl.BlockSpec((_BE, _CE), lambda i: (i, 0)),
            pl.BlockSpec((_CF, _H), lambda i: (0, 0)),
            pl.BlockSpec((_H, _H), lambda i: (0, 0)),
            pl.BlockSpec((_H, _H), lambda i: (0, 0)),
            pl.BlockSpec((_H, _CE * _D), lambda i: (0, 0)),
            pl.BlockSpec((_CE, _CE * _D), lambda i: (0, 0)),
        ],
        out_specs=pl.BlockSpec((_BE, _D), lambda i: (i, 0)),
        out_shape=jax.ShapeDtypeStruct((_E, _D), jnp.float32),
        compiler_params=pltpu.CompilerParams(dimension_semantics=("parallel",)),
    )(edge_feats, edge_attrs_real, W1, W2, W3, W4w, S)


def _idx_body(ei_ref, out_ref):
    out_ref[...] = ei_ref[...]


def _idx_call(edge_index_flat):
    return pl.pallas_call(
        _idx_body,
        out_shape=jax.ShapeDtypeStruct((2 * _E,), jnp.int32),
    )(edge_index_flat)


_NC = 2
_NS = 16
_NW = _NC * _NS
_CH = 128
_NCHUNKS = _E // _CH
_CPW = -(-_NCHUNKS // _NW)
_RPT = 632
_RPT_LAST = _N - 15 * _RPT
_RZB = 8


def _sc_body(h_hbm, acc_hbm, ei_hbm, out_hbm,
             sidx_v, ridx_v, hrows_v, arows_v, zbuf_v, msh, sem0, sem1):
    cid = lax.axis_index("c")
    sid = lax.axis_index("s")
    wid = sid * _NC + cid
    sems = (sem0, sem1)

    def _zfill(i, carry):
        r = i // (_D // 16)
        c = (i % (_D // 16)) * 16
        zbuf_v[r, pl.ds(c, 16)] = jnp.zeros((16,), jnp.float32)
        return carry

    lax.fori_loop(0, _RZB * (_D // 16), _zfill, 0)

    row0 = sid * _RPT
    nrows = jnp.where(sid == _NS - 1, _RPT_LAST, _RPT)

    def _zcopy(m, carry):
        pltpu.sync_copy(zbuf_v, msh.at[pl.ds(row0 + m * _RZB, _RZB)])
        return carry

    lax.fori_loop(0, nrows // _RZB, _zcopy, 0)

    def _issue(k, b):
        c = k * _NW + wid

        @pl.when(c < _NCHUNKS)
        def _():
            base = c * _CH
            pltpu.sync_copy(ei_hbm.at[pl.ds(base, _CH)], sidx_v.at[b])
            pltpu.sync_copy(ei_hbm.at[pl.ds(_E + base, _CH)], ridx_v.at[b])
            pltpu.async_copy(h_hbm.at[sidx_v.at[b]], hrows_v.at[b], sems[b])

    plsc.subcore_barrier()
    _issue(0, 0)

    def _pair(p, carry):
        for b in range(2):
            k = p * 2 + b
            c = k * _NW + wid

            @pl.when(c < _NCHUNKS)
            def _():
                pltpu.make_async_copy(h_hbm.at[pl.ds(0, _CH)], hrows_v.at[b],
                                      sems[b]).wait()
                _issue(k + 1, 1 - b)
                pltpu.sync_copy(acc_hbm.at[pl.ds(c * _CH, _CH)], arows_v)

                hb = hrows_v.at[b]
                ab = arows_v

                def _mul(q, carry2):
                    for rr in range(4):
                        r = q * 4 + rr
                        for j in range(_D // 16):
                            col = j * 16
                            hb[r, pl.ds(col, 16)] = (
                                hb[r, pl.ds(col, 16)] * ab[r, pl.ds(col, 16)])
                    return carry2

                lax.fori_loop(0, _CH // 4, _mul, 0)
                pltpu.sync_copy(hb, msh.at[ridx_v.at[b]], add=True)

        return carry

    lax.fori_loop(0, _CPW // 2, _pair, 0)
    plsc.subcore_barrier()

    @pl.when(sid < _NS - 1)
    def _():
        pltpu.sync_copy(msh.at[pl.ds(row0, _RPT)],
                        out_hbm.at[cid, pl.ds(row0, _RPT)])

    @pl.when(sid == _NS - 1)
    def _():
        pltpu.sync_copy(msh.at[pl.ds(row0, _RPT_LAST)],
                        out_hbm.at[cid, pl.ds(row0, _RPT_LAST)])


def _sc_call(h, acc, ei_flat):
    mesh = plsc.VectorSubcoreMesh(core_axis_name="c", subcore_axis_name="s")
    f = pl.kernel(
        _sc_body,
        mesh=mesh,
        out_type=jax.ShapeDtypeStruct((_NC, _N, _D), jnp.float32),
        scratch_types=[
            pltpu.VMEM((2, _CH), jnp.int32),
            pltpu.VMEM((2, _CH), jnp.int32),
            pltpu.VMEM((2, _CH, _D), jnp.float32),
            pltpu.VMEM((_CH, _D), jnp.float32),
            pltpu.VMEM((_RZB, _D), jnp.float32),
            pltpu.VMEM_SHARED((_N, _D), jnp.float32),
            pltpu.SemaphoreType.DMA,
            pltpu.SemaphoreType.DMA,
        ],
    )
    return f(h, acc, ei_flat)


def _post_body(mp_ref, nf_ref, na_ref, wlin_ref, wskip_ref, mr_ref, mi_ref):
    m = mp_ref[0] + mp_ref[1]
    nf = nf_ref[...]
    na = na_ref[...]
    sc = jnp.zeros((_BN, _D), jnp.float32)
    for v in range(_A):
        sc = sc + na[:, v:v + 1] * jnp.dot(nf, wskip_ref[:, v, :],
                                           preferred_element_type=jnp.float32)
    sc = sc * (1.0 / math.sqrt(_D * _A))
    s = 1.0 / (math.sqrt(_D) * 2.0 * _NUM_AVG_NEIGHBORS)
    mr = jnp.dot(m, wlin_ref[...], preferred_element_type=jnp.float32) * s + sc
    mr = jax.nn.silu(mr)
    mi = jax.nn.silu(jnp.dot(mr, wlin_ref[...],
                             preferred_element_type=jnp.float32) * s)
    mr_ref[...] = mr
    mi_ref[...] = mi


def _post_call(mp, node_feats, node_attrs, W_lin, W_skip):
    return pl.pallas_call(
        _post_body,
        grid=(_N // _BN,),
        in_specs=[
            pl.BlockSpec((_NC, _BN, _D), lambda i: (0, i, 0)),
            pl.BlockSpec((_BN, _D), lambda i: (i, 0)),
            pl.BlockSpec((_BN, _A), lambda i: (i, 0)),
            pl.BlockSpec((_D, _D), lambda i: (0, 0)),
            pl.BlockSpec((_D, _A, _D), lambda i: (0, 0, 0)),
        ],
        out_specs=[
            pl.BlockSpec((_BN, _D), lambda i: (i, 0)),
            pl.BlockSpec((_BN, _D), lambda i: (i, 0)),
        ],
        out_shape=[
            jax.ShapeDtypeStruct((_N, _D), jnp.float32),
            jax.ShapeDtypeStruct((_N, _D), jnp.float32),
        ],
        compiler_params=pltpu.CompilerParams(dimension_semantics=("parallel",)),
    )(mp, node_feats, node_attrs, W_lin, W_skip)


def kernel(node_attrs, node_feats, edge_attrs_real, edge_attrs_imag, edge_feats,
           edge_index, W_up, W1, W2, W3, W4, W_lin, W_skip):
    del edge_attrs_imag
    W4w = W4.reshape(_H, _D, _CE).transpose(0, 2, 1).reshape(_H, _CE * _D)
    S = jnp.kron(jnp.eye(_CE, dtype=jnp.float32),
                 jnp.ones((1, _D), jnp.float32))
    ei_flat = _idx_call(edge_index.reshape(2 * _E))
    h = _h_call(node_feats, W_up)
    acc = _edge_call(edge_feats, edge_attrs_real, W1, W2, W3, W4w, S)
    mp = _sc_call(h, acc, ei_flat)
    mr, mi = _post_call(mp, node_feats, node_attrs, W_lin, W_skip)
    return jnp.stack((mr, mi), axis=-1).reshape(_N, _D, 1, 2)


# speedup vs baseline: 1.0649x; 1.0617x over previous
import functools
import math

import jax
import jax.numpy as jnp
from jax import lax
from jax.experimental import pallas as pl
from jax.experimental.pallas import tpu as pltpu
from jax.experimental.pallas import tpu_sc as plsc

_N = 10000
_E = 160000
_D = 128
_A = 16
_CE = 4
_CF = 8
_H = 64
_NUM_AVG_NEIGHBORS = 16.0

_BN = 2000


def _h_body(nf_ref, wup_ref, h_ref):
    h_ref[...] = jnp.dot(nf_ref[...], wup_ref[...],
                         preferred_element_type=jnp.float32) * (1.0 / math.sqrt(_D))


def _h_call(node_feats, W_up):
    return pl.pallas_call(
        _h_body,
        grid=(_N // _BN,),
        in_specs=[
            pl.BlockSpec((_BN, _D), lambda i: (i, 0)),
            pl.BlockSpec((_D, _D), lambda i: (0, 0)),
        ],
        out_specs=pl.BlockSpec((_BN, _D), lambda i: (i, 0)),
        out_shape=jax.ShapeDtypeStruct((_N, _D), jnp.float32),
    )(node_feats, W_up)


_BE = 2000


def _edge_body(efr_ref, w1_ref, w2_ref, w3_ref, w4w_ref, s_ref, acc_ref):
    efr = efr_ref[...]
    ef = efr[:, 0:_CF]
    t = jax.nn.silu(jnp.dot(ef, w1_ref[...],
                            preferred_element_type=jnp.float32) * (1.0 / math.sqrt(_CF)))
    t = jax.nn.silu(jnp.dot(t, w2_ref[...],
                            preferred_element_type=jnp.float32) * (1.0 / math.sqrt(_H)))
    t = jax.nn.silu(jnp.dot(t, w3_ref[...],
                            preferred_element_type=jnp.float32) * (1.0 / math.sqrt(_H)))
    u = jnp.dot(t, w4w_ref[...], preferred_element_type=jnp.float32)
    er = efr[:, _CF:_CF + _CE]
    erb = jnp.dot(er, s_ref[...], preferred_element_type=jnp.float32)
    p = u * erb
    acc = p[:, 0:_D]
    for v in range(1, _CE):
        acc = acc + p[:, v * _D:(v + 1) * _D]
    acc_ref[...] = acc * (1.0 / (math.sqrt(_H) * math.sqrt(_CE)))


def _edge_call(efr, W1, W2, W3, W4w, S):
    return pl.pallas_call(
        _edge_body,
        grid=(_E // _BE,),
        in_specs=[
            pl.BlockSpec((_BE, _CF + _CE), lambda i: (i, 0)),
            pl.BlockSpec((_CF, _H), lambda i: (0, 0)),
            pl.BlockSpec((_H, _H), lambda i: (0, 0)),
            pl.BlockSpec((_H, _H), lambda i: (0, 0)),
            pl.BlockSpec((_H, _CE * _D), lambda i: (0, 0)),
            pl.BlockSpec((_CE, _CE * _D), lambda i: (0, 0)),
        ],
        out_specs=pl.BlockSpec((_BE, _D), lambda i: (i, 0)),
        out_shape=jax.ShapeDtypeStruct((_E, _D), jnp.float32),
        compiler_params=pltpu.CompilerParams(dimension_semantics=("parallel",)),
    )(efr, W1, W2, W3, W4w, S)


_NC = 2
_NS = 16
_NW = _NC * _NS
_CH = 128
_NCHUNKS = _E // _CH
_CPW = -(-_NCHUNKS // _NW)
_RPT = 632
_RPT_LAST = _N - 15 * _RPT
_RZB = 8


def _sc_body(h_hbm, acc_hbm, ei_hbm, out_hbm,
             sidx_v, ridx_v, hrows_v, arows_v, zbuf_v, msh, sem0, sem1):
    cid = lax.axis_index("c")
    sid = lax.axis_index("s")
    wid = sid * _NC + cid
    sems = (sem0, sem1)

    def _zfill(i, carry):
        r = i // (_D // 16)
        c = (i % (_D // 16)) * 16
        zbuf_v[r, pl.ds(c, 16)] = jnp.zeros((16,), jnp.float32)
        return carry

    lax.fori_loop(0, _RZB * (_D // 16), _zfill, 0)

    row0 = sid * _RPT
    nrows = jnp.where(sid == _NS - 1, _RPT_LAST, _RPT)

    def _zcopy(m, carry):
        pltpu.sync_copy(zbuf_v, msh.at[pl.ds(row0 + m * _RZB, _RZB)])
        return carry

    lax.fori_loop(0, nrows // _RZB, _zcopy, 0)

    def _issue(k, b):
        c = k * _NW + wid

        @pl.when(c < _NCHUNKS)
        def _():
            base = c * _CH
            pltpu.sync_copy(ei_hbm.at[pl.ds(base, _CH)], sidx_v.at[b])
            pltpu.sync_copy(ei_hbm.at[pl.ds(_E + base, _CH)], ridx_v.at[b])
            pltpu.async_copy(h_hbm.at[sidx_v.at[b]], hrows_v.at[b], sems[b])

    plsc.subcore_barrier()
    _issue(0, 0)

    def _pair(p, carry):
        for b in range(2):
            k = p * 2 + b
            c = k * _NW + wid

            @pl.when(c < _NCHUNKS)
            def _():
                pltpu.make_async_copy(h_hbm.at[pl.ds(0, _CH)], hrows_v.at[b],
                                      sems[b]).wait()
                _issue(k + 1, 1 - b)
                pltpu.sync_copy(acc_hbm.at[pl.ds(c * _CH, _CH)], arows_v)

                hb = hrows_v.at[b]
                ab = arows_v

                def _mul(q, carry2):
                    for rr in range(4):
                        r = q * 4 + rr
                        for j in range(_D // 16):
                            col = j * 16
                            hb[r, pl.ds(col, 16)] = (
                                hb[r, pl.ds(col, 16)] * ab[r, pl.ds(col, 16)])
                    return carry2

                lax.fori_loop(0, _CH // 4, _mul, 0)
                pltpu.sync_copy(hb, msh.at[ridx_v.at[b]], add=True)

        return carry

    lax.fori_loop(0, _CPW // 2, _pair, 0)
    plsc.subcore_barrier()

    @pl.when(sid < _NS - 1)
    def _():
        pltpu.sync_copy(msh.at[pl.ds(row0, _RPT)],
                        out_hbm.at[cid, pl.ds(row0, _RPT)])

    @pl.when(sid == _NS - 1)
    def _():
        pltpu.sync_copy(msh.at[pl.ds(row0, _RPT_LAST)],
                        out_hbm.at[cid, pl.ds(row0, _RPT_LAST)])


def _sc_call(h, acc, ei_flat):
    mesh = plsc.VectorSubcoreMesh(core_axis_name="c", subcore_axis_name="s")
    f = pl.kernel(
        _sc_body,
        mesh=mesh,
        out_type=jax.ShapeDtypeStruct((_NC, _N, _D), jnp.float32),
        scratch_types=[
            pltpu.VMEM((2, _CH), jnp.int32),
            pltpu.VMEM((2, _CH), jnp.int32),
            pltpu.VMEM((2, _CH, _D), jnp.float32),
            pltpu.VMEM((_CH, _D), jnp.float32),
            pltpu.VMEM((_RZB, _D), jnp.float32),
            pltpu.VMEM_SHARED((_N, _D), jnp.float32),
            pltpu.SemaphoreType.DMA,
            pltpu.SemaphoreType.DMA,
        ],
    )
    return f(h, acc, ei_flat)


def _post_body(mp_ref, nf_ref, na_ref, wlin_ref, wskip_ref, mr_ref, mi_ref):
    m = mp_ref[0] + mp_ref[1]
    nf = nf_ref[...]
    na = na_ref[...]
    sc = jnp.zeros((_BN, _D), jnp.float32)
    for v in range(_A):
        sc = sc + na[:, v:v + 1] * jnp.dot(nf, wskip_ref[:, v, :],
                                           preferred_element_type=jnp.float32)
    sc = sc * (1.0 / math.sqrt(_D * _A))
    s = 1.0 / (math.sqrt(_D) * 2.0 * _NUM_AVG_NEIGHBORS)
    mr = jnp.dot(m, wlin_ref[...], preferred_element_type=jnp.float32) * s + sc
    mr = jax.nn.silu(mr)
    mi = jax.nn.silu(jnp.dot(mr, wlin_ref[...],
                             preferred_element_type=jnp.float32) * s)
    mr_ref[...] = mr
    mi_ref[...] = mi


def _post_call(mp, node_feats, node_attrs, W_lin, W_skip):
    return pl.pallas_call(
        _post_body,
        grid=(_N // _BN,),
        in_specs=[
            pl.BlockSpec((_NC, _BN, _D), lambda i: (0, i, 0)),
            pl.BlockSpec((_BN, _D), lambda i: (i, 0)),
            pl.BlockSpec((_BN, _A), lambda i: (i, 0)),
            pl.BlockSpec((_D, _D), lambda i: (0, 0)),
            pl.BlockSpec((_D, _A, _D), lambda i: (0, 0, 0)),
        ],
        out_specs=[
            pl.BlockSpec((_BN, _D), lambda i: (i, 0)),
            pl.BlockSpec((_BN, _D), lambda i: (i, 0)),
        ],
        out_shape=[
            jax.ShapeDtypeStruct((_N, _D), jnp.float32),
            jax.ShapeDtypeStruct((_N, _D), jnp.float32),
        ],
        compiler_params=pltpu.CompilerParams(dimension_semantics=("parallel",)),
    )(mp, node_feats, node_attrs, W_lin, W_skip)


def kernel(node_attrs, node_feats, edge_attrs_real, edge_attrs_imag, edge_feats,
           edge_index, W_up, W1, W2, W3, W4, W_lin, W_skip):
    del edge_attrs_imag
    W4w = W4.reshape(_H, _D, _CE).transpose(0, 2, 1).reshape(_H, _CE * _D)
    S = jnp.kron(jnp.eye(_CE, dtype=jnp.float32),
                 jnp.ones((1, _D), jnp.float32))
    ei_flat = edge_index.reshape(2 * _E)
    efr = jnp.concatenate([edge_feats, edge_attrs_real], axis=1)
    h = _h_call(node_feats, W_up)
    acc = _edge_call(efr, W1, W2, W3, W4w, S)
    mp = _sc_call(h, acc, ei_flat)
    mr, mi = _post_call(mp, node_feats, node_attrs, W_lin, W_skip)
    return jnp.stack((mr, mi), axis=-1).reshape(_N, _D, 1, 2)
